# Initial kernel scaffold; baseline (speedup 1.0000x reference)
import jax
import jax.numpy as jnp
from jax.experimental import pallas as pl


def kernel(src_points, tgt_points, batch_indices):
    raise NotImplementedError("write your pallas kernel here")


# speedup vs baseline: 9.7143x; 9.7143x over previous
import functools

import jax
import jax.numpy as jnp
from jax import lax
from jax.experimental import pallas as pl
from jax.experimental.pallas import tpu as pltpu
from jax.experimental.pallas import tpu_sc as plsc

NSEG = 2048
NC = 2
NS = 16
NW = NC * NS
LANES = 16
CHUNK = 2048
TABLE = 16 * NSEG


def _pass1_call(src3, tgt3, idx):
  n = idx.shape[0]
  ppt = n // NW
  nsub = ppt // CHUNK
  ngrp = CHUNK // LANES
  mesh = plsc.VectorSubcoreMesh(core_axis_name="c", subcore_axis_name="s")

  @functools.partial(
      pl.kernel, mesh=mesh,
      compiler_params=pltpu.CompilerParams(needs_layout_passes=False),
      out_type=jax.ShapeDtypeStruct((NW, TABLE), jnp.float32),
      scratch_types=[
          pltpu.VMEM((CHUNK * 3,), jnp.float32),
          pltpu.VMEM((CHUNK * 3,), jnp.float32),
          pltpu.VMEM((CHUNK,), jnp.int32),
          pltpu.VMEM((TABLE,), jnp.float32),
      ],
  )
  def k(src_h, tgt_h, idx_h, out_h, sv, tv, iv, tab):
    wid = lax.axis_index("s") * NC + lax.axis_index("c")
    iota3 = lax.iota(jnp.int32, LANES) * 3
    ones = jnp.ones((LANES,), jnp.float32)

    def zero_body(i, carry):
      tab[pl.ds(i * LANES, LANES)] = jnp.zeros((LANES,), jnp.float32)
      return carry
    lax.fori_loop(0, TABLE // LANES, zero_body, 0)

    def sub_body(sub, carry):
      base = wid * ppt + sub * CHUNK
      pltpu.sync_copy(src_h.at[pl.ds(base * 3, CHUNK * 3)], sv)
      pltpu.sync_copy(tgt_h.at[pl.ds(base * 3, CHUNK * 3)], tv)
      pltpu.sync_copy(idx_h.at[pl.ds(base, CHUNK)], iv)

      def grp_body(g, c):
        rows3 = g * (LANES * 3) + iota3
        ivec = iv[pl.ds(g * LANES, LANES)]
        sx = plsc.load_gather(sv, [rows3])
        sy = plsc.load_gather(sv, [rows3 + 1])
        sz = plsc.load_gather(sv, [rows3 + 2])
        tx = plsc.load_gather(tv, [rows3])
        ty = plsc.load_gather(tv, [rows3 + 1])
        tz = plsc.load_gather(tv, [rows3 + 2])
        vals = (ones, sx, sy, sz, tx, ty, tz,
                sx * tx, sx * ty, sx * tz,
                sy * tx, sy * ty, sy * tz,
                sz * tx, sz * ty, sz * tz)
        for j, v in enumerate(vals):
          plsc.addupdate_scatter(tab, [ivec + j * NSEG], v)
        return c
      lax.fori_loop(0, ngrp, grp_body, 0)
      return carry
    lax.fori_loop(0, nsub, sub_body, 0)
    pltpu.sync_copy(tab, out_h.at[wid])

  return k(src3, tgt3, idx)


def _solve_call(partials):
  def body(p_ref, o_ref):
    s = jnp.sum(p_ref[...], axis=0)
    inv = 1.0 / jnp.maximum(s[0], 1.0)
    ss = (s[1], s[2], s[3])
    st = (s[4], s[5], s[6])
    ms = tuple(a * inv for a in ss)
    mt = tuple(a * inv for a in st)
    H = [[s[7 + 3 * a + b] - ss[a] * st[b] * inv for b in range(3)]
         for a in range(3)]
    (Sxx, Sxy, Sxz), (Syx, Syy, Syz), (Szx, Szy, Szz) = H
    n00 = Sxx + Syy + Szz
    n01 = Syz - Szy
    n02 = Szx - Sxz
    n03 = Sxy - Syx
    n11 = Sxx - Syy - Szz
    n12 = Sxy + Syx
    n13 = Szx + Sxz
    n22 = -Sxx + Syy - Szz
    n23 = Syz + Szy
    n33 = -Sxx - Syy + Szz
    fro2 = sum(H[a][b] * H[a][b] for a in range(3) for b in range(3))
    shift = jnp.sqrt(3.0 * fro2) + 1e-30
    B = [[n00 + shift, n01, n02, n03],
         [n01, n11 + shift, n12, n13],
         [n02, n12, n22 + shift, n23],
         [n03, n13, n23, n33 + shift]]
    for _ in range(16):
      C = [[sum(B[i][k] * B[k][j] for k in range(4)) for j in range(4)]
           for i in range(4)]
      invtr = 1.0 / jnp.maximum(C[0][0] + C[1][1] + C[2][2] + C[3][3], 1e-30)
      B = [[C[i][j] * invtr for j in range(4)] for i in range(4)]
    d = [B[i][i] for i in range(4)]
    m0 = (d[0] >= d[1]) & (d[0] >= d[2]) & (d[0] >= d[3])
    m1 = (d[1] >= d[2]) & (d[1] >= d[3])
    m2 = d[2] >= d[3]
    q = [jnp.where(m0, B[i][0],
         jnp.where(m1, B[i][1],
         jnp.where(m2, B[i][2], B[i][3]))) for i in range(4)]
    qn = 1.0 / jnp.sqrt(q[0] * q[0] + q[1] * q[1] + q[2] * q[2]
                        + q[3] * q[3] + 1e-30)
    w, x, y, z = (qi * qn for qi in q)
    r = [1.0 - 2.0 * (y * y + z * z), 2.0 * (x * y - w * z), 2.0 * (x * z + w * y),
         2.0 * (x * y + w * z), 1.0 - 2.0 * (x * x + z * z), 2.0 * (y * z - w * x),
         2.0 * (x * z - w * y), 2.0 * (y * z + w * x), 1.0 - 2.0 * (x * x + y * y)]
    t = [mt[a] - (r[3 * a] * ms[0] + r[3 * a + 1] * ms[1] + r[3 * a + 2] * ms[2])
         for a in range(3)]
    for j in range(9):
      o_ref[j, :] = r[j]
    for a in range(3):
      o_ref[9 + a, :] = t[a]
    zero = jnp.zeros((NSEG,), jnp.float32)
    for j in range(12, 16):
      o_ref[j, :] = zero

  return pl.pallas_call(
      body,
      out_shape=jax.ShapeDtypeStruct((16, NSEG), jnp.float32),
  )(partials)


def _apply_call(src3, idx, rt_flat):
  n = idx.shape[0]
  ppt = n // NW
  nsub = ppt // CHUNK
  ngrp = CHUNK // LANES
  mesh = plsc.VectorSubcoreMesh(core_axis_name="c", subcore_axis_name="s")

  @functools.partial(
      pl.kernel, mesh=mesh,
      compiler_params=pltpu.CompilerParams(needs_layout_passes=False),
      out_type=jax.ShapeDtypeStruct((n * 3,), jnp.float32),
      scratch_types=[
          pltpu.VMEM((CHUNK * 3,), jnp.float32),
          pltpu.VMEM((CHUNK,), jnp.int32),
          pltpu.VMEM((CHUNK * 3,), jnp.float32),
          pltpu.VMEM((12 * NSEG,), jnp.float32),
      ],
  )
  def k(src_h, idx_h, rt_h, out_h, sv, iv, ov, rtv):
    wid = lax.axis_index("s") * NC + lax.axis_index("c")
    iota3 = lax.iota(jnp.int32, LANES) * 3
    pltpu.sync_copy(rt_h, rtv)

    def sub_body(sub, carry):
      base = wid * ppt + sub * CHUNK
      pltpu.sync_copy(src_h.at[pl.ds(base * 3, CHUNK * 3)], sv)
      pltpu.sync_copy(idx_h.at[pl.ds(base, CHUNK)], iv)

      def grp_body(g, c):
        rows3 = g * (LANES * 3) + iota3
        ivec = iv[pl.ds(g * LANES, LANES)]
        sx = plsc.load_gather(sv, [rows3])
        sy = plsc.load_gather(sv, [rows3 + 1])
        sz = plsc.load_gather(sv, [rows3 + 2])
        coef = [plsc.load_gather(rtv, [ivec + j * NSEG]) for j in range(12)]
        ax = coef[0] * sx + coef[1] * sy + coef[2] * sz + coef[9]
        ay = coef[3] * sx + coef[4] * sy + coef[5] * sz + coef[10]
        az = coef[6] * sx + coef[7] * sy + coef[8] * sz + coef[11]
        plsc.store_scatter(ov, [rows3], ax)
        plsc.store_scatter(ov, [rows3 + 1], ay)
        plsc.store_scatter(ov, [rows3 + 2], az)
        return c
      lax.fori_loop(0, ngrp, grp_body, 0)
      pltpu.sync_copy(ov, out_h.at[pl.ds(base * 3, CHUNK * 3)])
      return carry
    lax.fori_loop(0, nsub, sub_body, 0)

  return k(src3, idx, rt_flat)


def kernel(src_points, tgt_points, batch_indices):
  n = src_points.shape[0]
  src3 = src_points.astype(jnp.float32).reshape(n * 3)
  tgt3 = tgt_points.astype(jnp.float32).reshape(n * 3)
  idx = batch_indices.astype(jnp.int32)
  partials = _pass1_call(src3, tgt3, idx)
  rt = _solve_call(partials.reshape(NW, 16, NSEG))
  aligned = _apply_call(src3, idx, rt[:12].reshape(12 * NSEG)).reshape(n, 3)
  R = jnp.transpose(rt[:9]).reshape(NSEG, 3, 3)
  t = jnp.transpose(rt[9:12])
  return (aligned, (R, t))
